# baseline (speedup 1.0000x reference)
import jax
import jax.numpy as jnp
from jax.experimental import pallas as pl

BATCH = 16384
MF_DIM = 16
MLP_DIM = 32


def _mlp_chain_body(umlpT_ref, mmlpT_ref, w1a_ref, w1b_ref, b1_ref,
                    w2_ref, b2_ref, w3_ref, b3_ref, h3_ref):
    h1 = jnp.maximum(
        jnp.dot(w1a_ref[...], umlpT_ref[...], preferred_element_type=jnp.float32)
        + jnp.dot(w1b_ref[...], mmlpT_ref[...], preferred_element_type=jnp.float32)
        + b1_ref[...], 0.0)
    h2 = jnp.maximum(
        jnp.dot(w2_ref[...], h1, preferred_element_type=jnp.float32)
        + b2_ref[...], 0.0)
    h3_ref[...] = jnp.maximum(
        jnp.dot(w3_ref[...], h2, preferred_element_type=jnp.float32)
        + b3_ref[...], 0.0)


def _final_body(umfT_ref, mmfT_ref, h3_ref, wfa_ref, wfb_ref, bf_ref,
                out_ref):
    gmf = umfT_ref[...] * mmfT_ref[...]
    fin = (jnp.dot(wfa_ref[...], gmf, preferred_element_type=jnp.float32)
           + jnp.dot(wfb_ref[...], h3_ref[...], preferred_element_type=jnp.float32)
           + bf_ref[0, 0])
    out_ref[...] = 5.0 * jax.nn.sigmoid(fin)


def _tc_mlp_chain(umlpT, mmlpT, w1a, w1b, b1, w2, b2, w3, b3):
    bk = 4096
    grid = (BATCH // bk,)
    full = lambda i: (0, 0)
    col = lambda i: (0, i)
    return pl.pallas_call(
        _mlp_chain_body,
        grid=grid,
        in_specs=[
            pl.BlockSpec((MLP_DIM, bk), col),
            pl.BlockSpec((MLP_DIM, bk), col),
            pl.BlockSpec((2 * MLP_DIM, MLP_DIM), full),
            pl.BlockSpec((2 * MLP_DIM, MLP_DIM), full),
            pl.BlockSpec((2 * MLP_DIM, 1), full),
            pl.BlockSpec((2 * MLP_DIM, 2 * MLP_DIM), full),
            pl.BlockSpec((2 * MLP_DIM, 1), full),
            pl.BlockSpec((MLP_DIM, 2 * MLP_DIM), full),
            pl.BlockSpec((MLP_DIM, 1), full),
        ],
        out_specs=pl.BlockSpec((MLP_DIM, bk), col),
        out_shape=jax.ShapeDtypeStruct((MLP_DIM, BATCH), jnp.float32),
    )(umlpT, mmlpT, w1a, w1b, b1, w2, b2, w3, b3)


def _tc_final(umfT, mmfT, h3T, wfa, wfb, bf):
    bk = 4096
    grid = (BATCH // bk,)
    full = lambda i: (0, 0)
    col = lambda i: (0, i)
    return pl.pallas_call(
        _final_body,
        grid=grid,
        in_specs=[
            pl.BlockSpec((MF_DIM, bk), col),
            pl.BlockSpec((MF_DIM, bk), col),
            pl.BlockSpec((MLP_DIM, bk), col),
            pl.BlockSpec((1, MF_DIM), full),
            pl.BlockSpec((1, MLP_DIM), full),
            pl.BlockSpec((1, 1), full),
        ],
        out_specs=pl.BlockSpec((1, bk), col),
        out_shape=jax.ShapeDtypeStruct((1, BATCH), jnp.float32),
    )(umfT, mmfT, h3T, wfa, wfb, bf)


def kernel(X, user_mf, movie_mf, user_mlp, movie_mlp,
           W1, b1, W2, b2, W3, b3, Wf, bf):
    uidx = X[:, 0:1]
    midx = X[:, 1:2]
    dn = jax.lax.GatherDimensionNumbers(
        offset_dims=(1,), collapsed_slice_dims=(0,), start_index_map=(0,))
    pib = jax.lax.GatherScatterMode.PROMISE_IN_BOUNDS

    def take(table, idx):
        return jax.lax.gather(table, idx, dn,
                              slice_sizes=(1, table.shape[1]), mode=pib)

    umlpT = take(user_mlp, uidx).T
    mmlpT = take(movie_mlp, midx).T
    umfT = take(user_mf, uidx).T
    mmfT = take(movie_mf, midx).T
    w1a = W1[:, :MLP_DIM]
    w1b = W1[:, MLP_DIM:]
    wfa = Wf[:, :MF_DIM]
    wfb = Wf[:, MF_DIM:]
    h3T = _tc_mlp_chain(umlpT, mmlpT, w1a, w1b, b1.reshape(-1, 1),
                        W2, b2.reshape(-1, 1), W3, b3.reshape(-1, 1))
    out = _tc_final(umfT, mmfT, h3T, wfa, wfb, bf.reshape(1, 1))
    return out.reshape(BATCH, 1)


# speedup vs baseline: 1.0038x; 1.0038x over previous
import jax
import jax.numpy as jnp
from jax.experimental import pallas as pl

BATCH = 16384
MF_DIM = 16
MLP_DIM = 32


def _mlp_chain_body(umlpT_ref, mmlpT_ref, w1a_ref, w1b_ref, b1_ref,
                    w2_ref, b2_ref, w3_ref, b3_ref, wfb_ref, bf_ref,
                    s2_ref):
    h1 = jnp.maximum(
        jnp.dot(w1a_ref[...], umlpT_ref[...], preferred_element_type=jnp.float32)
        + jnp.dot(w1b_ref[...], mmlpT_ref[...], preferred_element_type=jnp.float32)
        + b1_ref[...], 0.0)
    h2 = jnp.maximum(
        jnp.dot(w2_ref[...], h1, preferred_element_type=jnp.float32)
        + b2_ref[...], 0.0)
    h3 = jnp.maximum(
        jnp.dot(w3_ref[...], h2, preferred_element_type=jnp.float32)
        + b3_ref[...], 0.0)
    s2_ref[...] = (jnp.dot(wfb_ref[...], h3,
                           preferred_element_type=jnp.float32)
                   + bf_ref[0, 0])


def _final_body(umfT_ref, mmfT_ref, s2_ref, wfa_ref, out_ref):
    gmf = umfT_ref[...] * mmfT_ref[...]
    fin = (jnp.dot(wfa_ref[...], gmf, preferred_element_type=jnp.float32)
           + s2_ref[...])
    out_ref[...] = 5.0 * jax.nn.sigmoid(fin)


def _tc_mlp_chain(umlpT, mmlpT, w1a, w1b, b1, w2, b2, w3, b3, wfb, bf):
    bk = 4096
    grid = (BATCH // bk,)
    full = lambda i: (0, 0)
    col = lambda i: (0, i)
    return pl.pallas_call(
        _mlp_chain_body,
        grid=grid,
        in_specs=[
            pl.BlockSpec((MLP_DIM, bk), col),
            pl.BlockSpec((MLP_DIM, bk), col),
            pl.BlockSpec((2 * MLP_DIM, MLP_DIM), full),
            pl.BlockSpec((2 * MLP_DIM, MLP_DIM), full),
            pl.BlockSpec((2 * MLP_DIM, 1), full),
            pl.BlockSpec((2 * MLP_DIM, 2 * MLP_DIM), full),
            pl.BlockSpec((2 * MLP_DIM, 1), full),
            pl.BlockSpec((MLP_DIM, 2 * MLP_DIM), full),
            pl.BlockSpec((MLP_DIM, 1), full),
            pl.BlockSpec((1, MLP_DIM), full),
            pl.BlockSpec((1, 1), full),
        ],
        out_specs=pl.BlockSpec((1, bk), col),
        out_shape=jax.ShapeDtypeStruct((1, BATCH), jnp.float32),
    )(umlpT, mmlpT, w1a, w1b, b1, w2, b2, w3, b3, wfb, bf)


def _tc_final(umfT, mmfT, s2, wfa):
    bk = 4096
    grid = (BATCH // bk,)
    full = lambda i: (0, 0)
    col = lambda i: (0, i)
    return pl.pallas_call(
        _final_body,
        grid=grid,
        in_specs=[
            pl.BlockSpec((MF_DIM, bk), col),
            pl.BlockSpec((MF_DIM, bk), col),
            pl.BlockSpec((1, bk), col),
            pl.BlockSpec((1, MF_DIM), full),
        ],
        out_specs=pl.BlockSpec((1, bk), col),
        out_shape=jax.ShapeDtypeStruct((1, BATCH), jnp.float32),
    )(umfT, mmfT, s2, wfa)


def kernel(X, user_mf, movie_mf, user_mlp, movie_mlp,
           W1, b1, W2, b2, W3, b3, Wf, bf):
    uidx = X[:, 0:1]
    midx = X[:, 1:2]
    dn = jax.lax.GatherDimensionNumbers(
        offset_dims=(1,), collapsed_slice_dims=(0,), start_index_map=(0,))
    pib = jax.lax.GatherScatterMode.PROMISE_IN_BOUNDS

    def take(table, idx):
        return jax.lax.gather(table, idx, dn,
                              slice_sizes=(1, table.shape[1]), mode=pib)

    umlpT = take(user_mlp, uidx).T
    mmlpT = take(movie_mlp, midx).T
    umfT = take(user_mf, uidx).T
    mmfT = take(movie_mf, midx).T
    w1a = W1[:, :MLP_DIM]
    w1b = W1[:, MLP_DIM:]
    wfa = Wf[:, :MF_DIM]
    wfb = Wf[:, MF_DIM:]
    s2 = _tc_mlp_chain(umlpT, mmlpT, w1a, w1b, b1.reshape(-1, 1),
                       W2, b2.reshape(-1, 1), W3, b3.reshape(-1, 1),
                       wfb, bf.reshape(1, 1))
    out = _tc_final(umfT, mmfT, s2, wfa)
    return out.reshape(BATCH, 1)


# speedup vs baseline: 1.0182x; 1.0144x over previous
import jax
import jax.numpy as jnp
from jax.experimental import pallas as pl

BATCH = 16384
MF_DIM = 16
MLP_DIM = 32


def _mlp_chain_body(umlpT_ref, mmlpT_ref, w1a_ref, w1b_ref, b1_ref,
                    w2_ref, b2_ref, w3_ref, b3_ref, wfb_ref, bf_ref,
                    s2_ref):
    h1 = jnp.maximum(
        jnp.dot(w1a_ref[...], umlpT_ref[...], preferred_element_type=jnp.float32)
        + jnp.dot(w1b_ref[...], mmlpT_ref[...], preferred_element_type=jnp.float32)
        + b1_ref[...], 0.0)
    h2 = jnp.maximum(
        jnp.dot(w2_ref[...], h1, preferred_element_type=jnp.float32)
        + b2_ref[...], 0.0)
    h3 = jnp.maximum(
        jnp.dot(w3_ref[...], h2, preferred_element_type=jnp.float32)
        + b3_ref[...], 0.0)
    s2_ref[...] = (jnp.dot(wfb_ref[...], h3,
                           preferred_element_type=jnp.float32)
                   + bf_ref[0, 0])


def _final_body(umfT_ref, mmfT_ref, s2_ref, wfa_ref, out_ref):
    gmf = umfT_ref[...] * mmfT_ref[...]
    fin = (jnp.dot(wfa_ref[...], gmf, preferred_element_type=jnp.float32)
           + s2_ref[...])
    out_ref[...] = 5.0 * jax.nn.sigmoid(fin)


def _tc_mlp_chain(umlpT, mmlpT, w1a, w1b, b1, w2, b2, w3, b3, wfb, bf):
    bk = 4096
    grid = (BATCH // bk,)
    full = lambda i: (0, 0)
    col = lambda i: (0, i)
    return pl.pallas_call(
        _mlp_chain_body,
        grid=grid,
        in_specs=[
            pl.BlockSpec((MLP_DIM, bk), col),
            pl.BlockSpec((MLP_DIM, bk), col),
            pl.BlockSpec((2 * MLP_DIM, MLP_DIM), full),
            pl.BlockSpec((2 * MLP_DIM, MLP_DIM), full),
            pl.BlockSpec((2 * MLP_DIM, 1), full),
            pl.BlockSpec((2 * MLP_DIM, 2 * MLP_DIM), full),
            pl.BlockSpec((2 * MLP_DIM, 1), full),
            pl.BlockSpec((MLP_DIM, 2 * MLP_DIM), full),
            pl.BlockSpec((MLP_DIM, 1), full),
            pl.BlockSpec((1, MLP_DIM), full),
            pl.BlockSpec((1, 1), full),
        ],
        out_specs=pl.BlockSpec((1, bk), col),
        out_shape=jax.ShapeDtypeStruct((1, BATCH), jnp.float32),
    )(umlpT, mmlpT, w1a, w1b, b1, w2, b2, w3, b3, wfb, bf)


def _tc_final(umfT, mmfT, s2, wfa):
    bk = 8192
    grid = (BATCH // bk,)
    full = lambda i: (0, 0)
    col = lambda i: (0, i)
    return pl.pallas_call(
        _final_body,
        grid=grid,
        in_specs=[
            pl.BlockSpec((MF_DIM, bk), col),
            pl.BlockSpec((MF_DIM, bk), col),
            pl.BlockSpec((1, bk), col),
            pl.BlockSpec((1, MF_DIM), full),
        ],
        out_specs=pl.BlockSpec((1, bk), col),
        out_shape=jax.ShapeDtypeStruct((1, BATCH), jnp.float32),
    )(umfT, mmfT, s2, wfa)


def kernel(X, user_mf, movie_mf, user_mlp, movie_mlp,
           W1, b1, W2, b2, W3, b3, Wf, bf):
    uidx = X[:, 0:1]
    midx = X[:, 1:2]
    dn = jax.lax.GatherDimensionNumbers(
        offset_dims=(1,), collapsed_slice_dims=(0,), start_index_map=(0,))
    pib = jax.lax.GatherScatterMode.PROMISE_IN_BOUNDS

    def take(table, idx):
        return jax.lax.gather(table, idx, dn,
                              slice_sizes=(1, table.shape[1]), mode=pib)

    umlpT = take(user_mlp, uidx).T
    mmlpT = take(movie_mlp, midx).T
    umfT = take(user_mf, uidx).T
    mmfT = take(movie_mf, midx).T
    w1a = W1[:, :MLP_DIM]
    w1b = W1[:, MLP_DIM:]
    wfa = Wf[:, :MF_DIM]
    wfb = Wf[:, MF_DIM:]
    s2 = _tc_mlp_chain(umlpT, mmlpT, w1a, w1b, b1.reshape(-1, 1),
                       W2, b2.reshape(-1, 1), W3, b3.reshape(-1, 1),
                       wfb, bf.reshape(1, 1))
    out = _tc_final(umfT, mmfT, s2, wfa)
    return out.reshape(BATCH, 1)
